# baseline (speedup 1.0000x reference)
import functools

import jax
import jax.numpy as jnp
from jax import lax
from jax.experimental import pallas as pl
from jax.experimental.pallas import tpu as pltpu
from jax.experimental.pallas import tpu_sc as plsc

N = 10000
E = 160000
D = 256
H = 256
HD = 128
G = 64
OUT = 128

NC = 2
NS = 16
CH = 64
EP = 163840
NCHUNK = EP // (NS * CH)
KB = 16
NBLK = NCHUNK // KB
RING = 4
ZROWS = 632
AGG_ROWS = ZROWS * NS
DUMMY = 10016


def _sc_phase(x2, idx6, zeros, out, agg_sh, ibuf, rows, gsem, isem, c, s,
              kk):
    pltpu.async_copy(idx6.at[c, s, 0], ibuf.at[0], isem.at[0])
    pltpu.sync_copy(zeros, agg_sh.at[pl.ds(s * ZROWS, ZROWS)])
    plsc.subcore_barrier()

    pltpu.make_async_copy(idx6.at[c, s, 0], ibuf.at[0], isem.at[0]).wait()
    pltpu.async_copy(idx6.at[c, s, 1], ibuf.at[1], isem.at[1])
    for q in range(RING - 1):
        pltpu.async_copy(x2.at[ibuf.at[0, q, 0]], rows.at[q], gsem.at[q])

    def block(blk, carry):
        p = blk % 2
        for i in range(KB):
            b = i % RING
            pltpu.make_async_copy(x2.at[ibuf.at[p, i, 0]], rows.at[b],
                                  gsem.at[b]).wait()
            t = i + RING - 1
            if t < KB:
                pltpu.async_copy(x2.at[ibuf.at[p, t, 0]], rows.at[t % RING],
                                 gsem.at[t % RING])
            else:
                if i == KB - (RING - 1):
                    @pl.when(blk + 1 < NBLK)
                    def _():
                        pltpu.make_async_copy(idx6.at[c, s, blk + 1],
                                              ibuf.at[1 - p],
                                              isem.at[1 - p]).wait()

                @pl.when(blk + 1 < NBLK)
                def _():
                    pltpu.async_copy(x2.at[ibuf.at[1 - p, t - KB, 0]],
                                     rows.at[t % RING], gsem.at[t % RING])

            pltpu.sync_copy(rows.at[b], agg_sh.at[ibuf.at[p, i, 1]], add=True)
            if i == KB - 1:
                @pl.when(blk + 2 < NBLK)
                def _():
                    pltpu.async_copy(idx6.at[c, s, blk + 2], ibuf.at[p],
                                     isem.at[p])
        return carry

    lax.fori_loop(0, NBLK, block, 0)
    plsc.subcore_barrier()
    pltpu.sync_copy(
        agg_sh.at[pl.ds(s * ZROWS, ZROWS)],
        out.at[kk, pl.ds(c * AGG_ROWS + s * ZROWS, ZROWS)],
    )


def _sc_segsum2_body(x2a, x2b, idx6a, idx6b, zeros, out, agg_sh, ibuf, rows,
                     gsem, isem):
    c = lax.axis_index("c")
    s = lax.axis_index("s")
    _sc_phase(x2a, idx6a, zeros, out, agg_sh, ibuf, rows, gsem, isem, c, s, 0)
    plsc.subcore_barrier()
    _sc_phase(x2b, idx6b, zeros, out, agg_sh, ibuf, rows, gsem, isem, c, s, 1)


_sc_segsum2 = functools.partial(
    pl.kernel,
    out_type=jax.ShapeDtypeStruct((2, 2 * AGG_ROWS, HD), jnp.float32),
    mesh=plsc.VectorSubcoreMesh(core_axis_name="c", subcore_axis_name="s"),
    scratch_types=[
        pltpu.VMEM_SHARED((AGG_ROWS, HD), jnp.float32),
        pltpu.VMEM((2, KB, 2, CH), jnp.int32),
        pltpu.VMEM((RING, CH, HD), jnp.float32),
        pltpu.SemaphoreType.DMA((RING,)),
        pltpu.SemaphoreType.DMA((2,)),
    ],
)(_sc_segsum2_body)


def _tc_mlp_body(x2, agg2, w1, b1, gam, bet, w2, b2, out):
    x = jnp.concatenate([x2[:N, :], x2[N:, :]], axis=1)
    a = jnp.concatenate([agg2[:N, :], agg2[AGG_ROWS:AGG_ROWS + N, :]], axis=1)
    h = jnp.dot(x + a, w1[:], preferred_element_type=jnp.float32) + b1[:]
    mu = jnp.mean(h, axis=0, keepdims=True)
    dcen = h - mu
    var = jnp.mean(dcen * dcen, axis=0, keepdims=True)
    hn = dcen * lax.rsqrt(var + 1e-5) * gam[:] + bet[:]
    hn = jnp.maximum(hn, 0.0)
    h2 = jnp.dot(hn, w2[:], preferred_element_type=jnp.float32) + b2[:]
    h2 = jnp.maximum(h2, 0.0)
    out[:N, :] = h2[:, :HD]
    out[N:, :] = h2[:, HD:]


_tc_mlp = pl.pallas_call(
    _tc_mlp_body,
    out_shape=jax.ShapeDtypeStruct((2 * N, HD), jnp.float32),
)


def _tc_mlp_last_body(x2, agg2, w1, b1, gam, bet, w2, b2, bat, out):
    x = jnp.concatenate([x2[:N, :], x2[N:, :]], axis=1)
    a = jnp.concatenate([agg2[:N, :], agg2[AGG_ROWS:AGG_ROWS + N, :]], axis=1)
    h = jnp.dot(x + a, w1[:], preferred_element_type=jnp.float32) + b1[:]
    mu = jnp.mean(h, axis=0, keepdims=True)
    dcen = h - mu
    var = jnp.mean(dcen * dcen, axis=0, keepdims=True)
    hn = dcen * lax.rsqrt(var + 1e-5) * gam[:] + bet[:]
    hn = jnp.maximum(hn, 0.0)
    h2 = jnp.dot(hn, w2[:], preferred_element_type=jnp.float32) + b2[:]
    h2 = jnp.maximum(h2, 0.0)
    gids = lax.broadcasted_iota(jnp.int32, (G, N), 0)
    oh = jnp.where(bat[:] == gids, 1.0, 0.0)
    out[:, :] = jnp.dot(oh, h2, preferred_element_type=jnp.float32)


_tc_mlp_last = pl.pallas_call(
    _tc_mlp_last_body,
    out_shape=jax.ShapeDtypeStruct((G, H), jnp.float32),
)


def _tc_final_body(pa, pb, wf1, bf1, wf2, bf2, out):
    z = jnp.concatenate([pa[:], pb[:]], axis=1)
    h = jnp.maximum(jnp.dot(z, wf1[:], preferred_element_type=jnp.float32) + bf1[:], 0.0)
    out[:, :] = jnp.dot(h, wf2[:], preferred_element_type=jnp.float32) + bf2[:]


_tc_final = pl.pallas_call(
    _tc_final_body,
    out_shape=jax.ShapeDtypeStruct((G, OUT), jnp.float32),
)


def _to_planes(x):
    return x.reshape(N, 2, HD).swapaxes(0, 1).reshape(2 * N, HD)


def _prep_edges(ei):
    src = ei[0]
    dst = ei[1]
    pad = EP - E
    srcp = jnp.concatenate([src, jnp.zeros((pad,), jnp.int32)])
    dstp = jnp.concatenate([dst, jnp.full((pad,), DUMMY, jnp.int32)])
    src2 = jnp.stack([srcp, srcp + N]).reshape(NC, NS, NBLK, KB, 1, CH)
    dst5 = jnp.broadcast_to(dstp.reshape(1, NS, NBLK, KB, 1, CH),
                            src2.shape)
    return jnp.concatenate([src2, dst5], axis=4)


def kernel(x0, x1, edge_index0, batch0, edge_index1, batch1, W1, b1, gamma,
           beta, W2, b2, Wf1, bf1, Wf2, bf2):
    zeros = jnp.zeros((ZROWS, HD), jnp.float32)
    idx = [_prep_edges(edge_index0), _prep_edges(edge_index1)]
    xs = [_to_planes(x0), _to_planes(x1)]
    bats = (batch0.reshape(1, N), batch1.reshape(1, N))
    pooled = [None, None]
    for l in range(3):
        agg2 = _sc_segsum2(xs[0], xs[1], idx[0], idx[1], zeros)
        aggs = [agg2[0], agg2[1]]
        for k in range(2):
            args = (xs[k], aggs[k], W1[k, l], b1[k, l].reshape(1, H),
                    gamma[k, l].reshape(1, H), beta[k, l].reshape(1, H),
                    W2[k, l], b2[k, l].reshape(1, H))
            if l < 2:
                xs[k] = _tc_mlp(*args)
            else:
                pooled[k] = _tc_mlp_last(*args, bats[k])
    return _tc_final(pooled[0], pooled[1],
                     Wf1, bf1.reshape(1, H), Wf2, bf2.reshape(1, OUT))


# speedup vs baseline: 1.0400x; 1.0400x over previous
import functools

import jax
import jax.numpy as jnp
from jax import lax
from jax.experimental import pallas as pl
from jax.experimental.pallas import tpu as pltpu
from jax.experimental.pallas import tpu_sc as plsc

N = 10000
E = 160000
D = 256
H = 256
HD = 128
G = 64
OUT = 128

NC = 2
NS = 16
CH = 64
EP = 163840
NCHUNK = EP // (NS * CH)
KB = 16
NBLK = NCHUNK // KB
RING = 4
ZROWS = 632
AGG_ROWS = ZROWS * NS
DUMMY = 10016


def _sc_segsum_body(x2, idx6, zeros, out, agg_sh, ibuf, rows, gsem, isem):
    c = lax.axis_index("c")
    s = lax.axis_index("s")
    pltpu.async_copy(idx6.at[c, s, 0], ibuf.at[0], isem.at[0])
    pltpu.sync_copy(zeros, agg_sh.at[pl.ds(s * ZROWS, ZROWS)])
    plsc.subcore_barrier()

    pltpu.make_async_copy(idx6.at[c, s, 0], ibuf.at[0], isem.at[0]).wait()
    pltpu.async_copy(idx6.at[c, s, 1], ibuf.at[1], isem.at[1])
    for q in range(RING - 1):
        pltpu.async_copy(x2.at[ibuf.at[0, q, 0]], rows.at[q], gsem.at[q])

    def block(blk, carry):
        p = blk % 2
        for i in range(KB):
            b = i % RING
            pltpu.make_async_copy(x2.at[ibuf.at[p, i, 0]], rows.at[b],
                                  gsem.at[b]).wait()
            t = i + RING - 1
            if t < KB:
                pltpu.async_copy(x2.at[ibuf.at[p, t, 0]], rows.at[t % RING],
                                 gsem.at[t % RING])
            else:
                if i == KB - (RING - 1):
                    @pl.when(blk + 1 < NBLK)
                    def _():
                        pltpu.make_async_copy(idx6.at[c, s, blk + 1],
                                              ibuf.at[1 - p],
                                              isem.at[1 - p]).wait()

                @pl.when(blk + 1 < NBLK)
                def _():
                    pltpu.async_copy(x2.at[ibuf.at[1 - p, t - KB, 0]],
                                     rows.at[t % RING], gsem.at[t % RING])

            pltpu.sync_copy(rows.at[b], agg_sh.at[ibuf.at[p, i, 1]], add=True)
            if i == KB - 1:
                @pl.when(blk + 2 < NBLK)
                def _():
                    pltpu.async_copy(idx6.at[c, s, blk + 2], ibuf.at[p],
                                     isem.at[p])
        return carry

    lax.fori_loop(0, NBLK, block, 0)
    plsc.subcore_barrier()
    pltpu.sync_copy(
        agg_sh.at[pl.ds(s * ZROWS, ZROWS)],
        out.at[pl.ds(c * AGG_ROWS + s * ZROWS, ZROWS)],
    )


_sc_segsum = functools.partial(
    pl.kernel,
    out_type=jax.ShapeDtypeStruct((2 * AGG_ROWS, HD), jnp.float32),
    mesh=plsc.VectorSubcoreMesh(core_axis_name="c", subcore_axis_name="s"),
    scratch_types=[
        pltpu.VMEM_SHARED((AGG_ROWS, HD), jnp.float32),
        pltpu.VMEM((2, KB, 2, CH), jnp.int32),
        pltpu.VMEM((RING, CH, HD), jnp.float32),
        pltpu.SemaphoreType.DMA((RING,)),
        pltpu.SemaphoreType.DMA((2,)),
    ],
)(_sc_segsum_body)


def _tc_mlp_body(x2, agg2, w1, b1, gam, bet, w2, b2, out):
    x = jnp.concatenate([x2[:N, :], x2[N:, :]], axis=1)
    a = jnp.concatenate([agg2[:N, :], agg2[AGG_ROWS:AGG_ROWS + N, :]], axis=1)
    h = jnp.dot(x + a, w1[:], preferred_element_type=jnp.float32) + b1[:]
    mu = jnp.mean(h, axis=0, keepdims=True)
    dcen = h - mu
    var = jnp.mean(dcen * dcen, axis=0, keepdims=True)
    hn = dcen * lax.rsqrt(var + 1e-5) * gam[:] + bet[:]
    hn = jnp.maximum(hn, 0.0)
    h2 = jnp.dot(hn, w2[:], preferred_element_type=jnp.float32) + b2[:]
    h2 = jnp.maximum(h2, 0.0)
    out[:N, :] = h2[:, :HD]
    out[N:, :] = h2[:, HD:]


_tc_mlp = pl.pallas_call(
    _tc_mlp_body,
    out_shape=jax.ShapeDtypeStruct((2 * N, HD), jnp.float32),
)


def _tc_mlp_last_body(x2, agg2, w1, b1, gam, bet, w2, b2, bat, out):
    x = jnp.concatenate([x2[:N, :], x2[N:, :]], axis=1)
    a = jnp.concatenate([agg2[:N, :], agg2[AGG_ROWS:AGG_ROWS + N, :]], axis=1)
    h = jnp.dot(x + a, w1[:], preferred_element_type=jnp.float32) + b1[:]
    mu = jnp.mean(h, axis=0, keepdims=True)
    dcen = h - mu
    var = jnp.mean(dcen * dcen, axis=0, keepdims=True)
    hn = dcen * lax.rsqrt(var + 1e-5) * gam[:] + bet[:]
    hn = jnp.maximum(hn, 0.0)
    h2 = jnp.dot(hn, w2[:], preferred_element_type=jnp.float32) + b2[:]
    h2 = jnp.maximum(h2, 0.0)
    gids = lax.broadcasted_iota(jnp.int32, (G, N), 0)
    oh = jnp.where(bat[:] == gids, 1.0, 0.0)
    out[:, :] = jnp.dot(oh, h2, preferred_element_type=jnp.float32)


_tc_mlp_last = pl.pallas_call(
    _tc_mlp_last_body,
    out_shape=jax.ShapeDtypeStruct((G, H), jnp.float32),
)


def _tc_final_body(pa, pb, wf1, bf1, wf2, bf2, out):
    z = jnp.concatenate([pa[:], pb[:]], axis=1)
    h = jnp.maximum(jnp.dot(z, wf1[:], preferred_element_type=jnp.float32) + bf1[:], 0.0)
    out[:, :] = jnp.dot(h, wf2[:], preferred_element_type=jnp.float32) + bf2[:]


_tc_final = pl.pallas_call(
    _tc_final_body,
    out_shape=jax.ShapeDtypeStruct((G, OUT), jnp.float32),
)


def _to_planes(x):
    return x.reshape(N, 2, HD).swapaxes(0, 1).reshape(2 * N, HD)


def _prep_edges(ei):
    src = ei[0]
    dst = ei[1]
    pad = EP - E
    srcp = jnp.concatenate([src, jnp.zeros((pad,), jnp.int32)])
    dstp = jnp.concatenate([dst, jnp.full((pad,), DUMMY, jnp.int32)])
    src2 = jnp.stack([srcp, srcp + N]).reshape(NC, NS, NBLK, KB, 1, CH)
    dst5 = jnp.broadcast_to(dstp.reshape(1, NS, NBLK, KB, 1, CH),
                            src2.shape)
    return jnp.concatenate([src2, dst5], axis=4)


def kernel(x0, x1, edge_index0, batch0, edge_index1, batch1, W1, b1, gamma,
           beta, W2, b2, Wf1, bf1, Wf2, bf2):
    zeros = jnp.zeros((ZROWS, HD), jnp.float32)
    idx = [_prep_edges(edge_index0), _prep_edges(edge_index1)]
    xs = [_to_planes(x0), _to_planes(x1)]
    bats = (batch0.reshape(1, N), batch1.reshape(1, N))
    pooled = [None, None]
    for l in range(3):
        aggs = [None, None]
        for k in range(2):
            aggs[k] = _sc_segsum(xs[k], idx[k], zeros)
        for k in range(2):
            args = (xs[k], aggs[k], W1[k, l], b1[k, l].reshape(1, H),
                    gamma[k, l].reshape(1, H), beta[k, l].reshape(1, H),
                    W2[k, l], b2[k, l].reshape(1, H))
            if l < 2:
                xs[k] = _tc_mlp(*args)
            else:
                pooled[k] = _tc_mlp_last(*args, bats[k])
    return _tc_final(pooled[0], pooled[1],
                     Wf1, bf1.reshape(1, H), Wf2, bf2.reshape(1, OUT))
